# Initial kernel scaffold; baseline (speedup 1.0000x reference)
import jax
import jax.numpy as jnp
from jax.experimental import pallas as pl


def kernel(x, edge_index, cw1, cb1, cg1, cbe1, cw2, cb2, bng, bnb, fcw, fcb):
    raise NotImplementedError("write your pallas kernel here")


# speedup vs baseline: 5.1543x; 5.1543x over previous
import functools

import jax
import jax.numpy as jnp
from jax import lax
from jax.experimental import pallas as pl
from jax.experimental.pallas import tpu as pltpu
from jax.experimental.pallas import tpu_sc as plsc

_N = 10000
_E = 320000
_D = 128
_C = 16
_L = 3

_NPAD = 10240
_CH = 128
_NCHUNK = _E // _CH
_NC = 2
_NS = 16
_NW = _NC * _NS
_CPW = -(-_NCHUNK // _NW)

_BLK = 1000


def _seg_sum_sc(h, src, dst, zrows):
    mesh = plsc.VectorSubcoreMesh(core_axis_name="c", subcore_axis_name="s")

    @functools.partial(
        pl.kernel,
        out_type=jax.ShapeDtypeStruct((_NC, _NPAD, _D), jnp.float32),
        mesh=mesh,
        scratch_types=[
            pltpu.VMEM((_CH,), jnp.int32),
            pltpu.VMEM((_CH,), jnp.int32),
            pltpu.VMEM((_CH, _D), jnp.float32),
            pltpu.VMEM_SHARED((_NPAD, _D), jnp.float32),
            pltpu.SemaphoreType.DMA,
        ],
    )
    def seg_kernel(h_hbm, src_hbm, dst_hbm, z_hbm, out_hbm,
                   src_v, dst_v, rows_v, acc, sem):
        cid = lax.axis_index("c")
        sid = lax.axis_index("s")
        wid = sid * _NC + cid

        pltpu.sync_copy(z_hbm, acc.at[pl.ds(sid * 640, 640)])
        plsc.subcore_barrier()

        def body(k, carry):
            c = wid + k * _NW

            @pl.when(c < _NCHUNK)
            def _():
                base = c * _CH
                pltpu.sync_copy(src_hbm.at[pl.ds(base, _CH)], src_v)
                pltpu.sync_copy(dst_hbm.at[pl.ds(base, _CH)], dst_v)
                pltpu.async_copy(h_hbm.at[src_v], rows_v, sem).wait()
                pltpu.sync_copy(rows_v, acc.at[dst_v], add=True)

            return carry

        lax.fori_loop(0, _CPW, body, 0)
        plsc.subcore_barrier()

        pltpu.sync_copy(acc.at[pl.ds(sid * 640, 640)],
                        out_hbm.at[cid, pl.ds(sid * 640, 640)])

    return seg_kernel(h, src, dst, zrows)


def _mm1_body(h_ref, a_ref, w_ref, b_ref, z_ref, st_ref):
    u = h_ref[...] + a_ref[0] + a_ref[1]
    z = jnp.dot(u, w_ref[...], preferred_element_type=jnp.float32) + b_ref[...]
    z_ref[...] = z

    @pl.when(pl.program_id(0) == 0)
    def _():
        st_ref[...] = jnp.zeros_like(st_ref)

    st_ref[0:1, :] += jnp.sum(z, axis=0, keepdims=True)
    st_ref[1:2, :] += jnp.sum(z * z, axis=0, keepdims=True)


def _mm2_body(z_ref, st_ref, g_ref, be_ref, w_ref, b_ref, z2_ref, st2_ref):
    mean = st_ref[0:1, :] * (1.0 / _N)
    var = st_ref[1:2, :] * (1.0 / _N) - mean * mean
    scale = g_ref[...] * lax.rsqrt(var + 1e-5)
    shift = be_ref[...] - mean * scale
    a = jnp.maximum(z_ref[...] * scale + shift, 0.0)
    z2 = jnp.dot(a, w_ref[...], preferred_element_type=jnp.float32) + b_ref[...]
    z2_ref[...] = z2

    @pl.when(pl.program_id(0) == 0)
    def _():
        st2_ref[...] = jnp.zeros_like(st2_ref)

    st2_ref[0:1, :] += jnp.sum(z2, axis=0, keepdims=True)
    st2_ref[1:2, :] += jnp.sum(z2 * z2, axis=0, keepdims=True)


def _bnrelu_body(z_ref, st_ref, g_ref, be_ref, h_ref):
    mean = st_ref[0:1, :] * (1.0 / _N)
    var = st_ref[1:2, :] * (1.0 / _N) - mean * mean
    scale = g_ref[...] * lax.rsqrt(var + 1e-5)
    shift = be_ref[...] - mean * scale
    h_ref[...] = jnp.maximum(z_ref[...] * scale + shift, 0.0)


def _heads_body(x_ref, h1_ref, h2_ref, h3_ref, fw_ref, fb_ref, o_ref):
    logits = jnp.dot(x_ref[...], fw_ref[0], preferred_element_type=jnp.float32)
    logits += jnp.dot(h1_ref[...], fw_ref[1], preferred_element_type=jnp.float32)
    logits += jnp.dot(h2_ref[...], fw_ref[2], preferred_element_type=jnp.float32)
    logits += jnp.dot(h3_ref[...], fw_ref[3], preferred_element_type=jnp.float32)
    logits += jnp.sum(fb_ref[...], axis=0, keepdims=True)
    m = jnp.max(logits, axis=-1, keepdims=True)
    e = jnp.exp(logits - m)
    o_ref[...] = logits - m - jnp.log(jnp.sum(e, axis=-1, keepdims=True))


def _row_spec():
    return pl.BlockSpec((_BLK, _D), lambda i: (i, 0))


def _full_spec(shape):
    nd = len(shape)
    return pl.BlockSpec(shape, lambda i: (0,) * nd)


def _mm1(h, agg2, w, b):
    return pl.pallas_call(
        _mm1_body,
        grid=(_N // _BLK,),
        in_specs=[
            _row_spec(),
            pl.BlockSpec((_NC, _BLK, _D), lambda i: (0, i, 0)),
            _full_spec((_D, _D)),
            _full_spec((1, _D)),
        ],
        out_specs=[_row_spec(), _full_spec((8, _D))],
        out_shape=[
            jax.ShapeDtypeStruct((_N, _D), jnp.float32),
            jax.ShapeDtypeStruct((8, _D), jnp.float32),
        ],
    )(h, agg2, w, b)


def _mm2(z, st, g, be, w, b):
    return pl.pallas_call(
        _mm2_body,
        grid=(_N // _BLK,),
        in_specs=[
            _row_spec(),
            _full_spec((8, _D)),
            _full_spec((1, _D)),
            _full_spec((1, _D)),
            _full_spec((_D, _D)),
            _full_spec((1, _D)),
        ],
        out_specs=[_row_spec(), _full_spec((8, _D))],
        out_shape=[
            jax.ShapeDtypeStruct((_N, _D), jnp.float32),
            jax.ShapeDtypeStruct((8, _D), jnp.float32),
        ],
    )(z, st, g, be, w, b)


def _bnrelu(z, st, g, be):
    return pl.pallas_call(
        _bnrelu_body,
        grid=(_N // _BLK,),
        in_specs=[
            _row_spec(),
            _full_spec((8, _D)),
            _full_spec((1, _D)),
            _full_spec((1, _D)),
        ],
        out_specs=_row_spec(),
        out_shape=jax.ShapeDtypeStruct((_N, _D), jnp.float32),
    )(z, st, g, be)


def _heads(x, h1, h2, h3, fw, fb):
    return pl.pallas_call(
        _heads_body,
        grid=(_N // _BLK,),
        in_specs=[
            _row_spec(),
            _row_spec(),
            _row_spec(),
            _row_spec(),
            _full_spec((_L + 1, _D, _C)),
            _full_spec((_L + 1, _C)),
        ],
        out_specs=pl.BlockSpec((_BLK, _C), lambda i: (i, 0)),
        out_shape=jax.ShapeDtypeStruct((_N, _C), jnp.float32),
    )(x, h1, h2, h3, fw, fb)


def kernel(x, edge_index, cw1, cb1, cg1, cbe1, cw2, cb2, bng, bnb, fcw, fcb):
    src = edge_index[0]
    dst = edge_index[1]
    zrows = jnp.zeros((640, _D), jnp.float32)

    h = x
    outs = [x]
    for l in range(_L):
        agg2 = _seg_sum_sc(h, src, dst, zrows)
        z1, st1 = _mm1(h, agg2, cw1[l], cb1[l].reshape(1, _D))
        z2, st2 = _mm2(z1, st1, cg1[l].reshape(1, _D), cbe1[l].reshape(1, _D),
                       cw2[l], cb2[l].reshape(1, _D))
        h = _bnrelu(z2, st2, bng[l].reshape(1, _D), bnb[l].reshape(1, _D))
        outs.append(h)

    out = _heads(outs[0], outs[1], outs[2], outs[3], fcw, fcb)
    return (out, 0)


# speedup vs baseline: 9.2707x; 1.7986x over previous
import functools

import jax
import jax.numpy as jnp
from jax import lax
from jax.experimental import pallas as pl
from jax.experimental.pallas import tpu as pltpu
from jax.experimental.pallas import tpu_sc as plsc

_N = 10000
_E = 320000
_D = 128
_C = 16
_L = 3

_NPAD = 10240
_CH = 128
_NC = 2
_NS = 16
_NW = _NC * _NS
_SLAB = 16
_NSLAB = 5
_CPW = _SLAB * _NSLAB
_EPAD = _NW * _CPW * _CH

_BLK = 1000


def _seg_sum_sc(h, src, dst, zrows):
    mesh = plsc.VectorSubcoreMesh(core_axis_name="c", subcore_axis_name="s")

    @functools.partial(
        pl.kernel,
        out_type=jax.ShapeDtypeStruct((_NC, _NPAD, _D), jnp.float32),
        mesh=mesh,
        scratch_types=[
            pltpu.VMEM((_SLAB, _CH), jnp.int32),
            pltpu.VMEM((_SLAB, _CH), jnp.int32),
            pltpu.VMEM((_SLAB, _CH), jnp.int32),
            pltpu.VMEM((_SLAB, _CH), jnp.int32),
            pltpu.VMEM((_CH, _D), jnp.float32),
            pltpu.VMEM((_CH, _D), jnp.float32),
            pltpu.VMEM_SHARED((_NPAD, _D), jnp.float32),
            pltpu.SemaphoreType.DMA,
            pltpu.SemaphoreType.DMA,
            pltpu.SemaphoreType.DMA,
        ],
    )
    def seg_kernel(h_hbm, src_hbm, dst_hbm, z_hbm, out_hbm,
                   src_s0, src_s1, dst_s0, dst_s1, rows0, rows1, acc,
                   semg0, semg1, semi):
        cid = lax.axis_index("c")
        sid = lax.axis_index("s")
        wid = sid * _NC + cid

        pltpu.async_copy(src_hbm.at[wid, 0], src_s0, semi)
        pltpu.async_copy(dst_hbm.at[wid, 0], dst_s0, semi)
        pltpu.sync_copy(z_hbm, acc.at[pl.ds(sid * 640, 640)])
        plsc.subcore_barrier()

        idx_bufs = [(src_s0, dst_s0), (src_s1, dst_s1)]
        for s in range(_NSLAB):
            ss, ds_ = idx_bufs[s % 2]
            ns, nd = idx_bufs[(s + 1) % 2]
            pltpu.make_async_copy(src_hbm.at[wid, s], ss, semi).wait()
            pltpu.make_async_copy(dst_hbm.at[wid, s], ds_, semi).wait()
            if s + 1 < _NSLAB:
                pltpu.async_copy(src_hbm.at[wid, s + 1], ns, semi)
                pltpu.async_copy(dst_hbm.at[wid, s + 1], nd, semi)

            pltpu.async_copy(h_hbm.at[ss.at[0]], rows0, semg0)

            def body(g, carry, ss=ss, ds_=ds_):
                j = 2 * g
                pltpu.async_copy(h_hbm.at[ss.at[j + 1]], rows1, semg1)
                pltpu.make_async_copy(h_hbm.at[ss.at[j]], rows0, semg0).wait()
                pltpu.sync_copy(rows0, acc.at[ds_.at[j]], add=True)

                @pl.when(g < _SLAB // 2 - 1)
                def _():
                    pltpu.async_copy(h_hbm.at[ss.at[j + 2]], rows0, semg0)

                pltpu.make_async_copy(h_hbm.at[ss.at[j + 1]], rows1,
                                      semg1).wait()
                pltpu.sync_copy(rows1, acc.at[ds_.at[j + 1]], add=True)
                return carry

            lax.fori_loop(0, _SLAB // 2, body, 0)

        plsc.subcore_barrier()

        pltpu.sync_copy(acc.at[pl.ds(sid * 640, 640)],
                        out_hbm.at[cid, pl.ds(sid * 640, 640)])

    return seg_kernel(h, src, dst, zrows)


def _mm1_body(h_ref, a_ref, w_ref, b_ref, z_ref, st_ref):
    u = h_ref[...] + a_ref[0] + a_ref[1]
    z = jnp.dot(u, w_ref[...], preferred_element_type=jnp.float32) + b_ref[...]
    z_ref[...] = z

    @pl.when(pl.program_id(0) == 0)
    def _():
        st_ref[...] = jnp.zeros_like(st_ref)

    st_ref[0:1, :] += jnp.sum(z, axis=0, keepdims=True)
    st_ref[1:2, :] += jnp.sum(z * z, axis=0, keepdims=True)


def _mm2_body(z_ref, st_ref, g_ref, be_ref, w_ref, b_ref, z2_ref, st2_ref):
    mean = st_ref[0:1, :] * (1.0 / _N)
    var = st_ref[1:2, :] * (1.0 / _N) - mean * mean
    scale = g_ref[...] * lax.rsqrt(var + 1e-5)
    shift = be_ref[...] - mean * scale
    a = jnp.maximum(z_ref[...] * scale + shift, 0.0)
    z2 = jnp.dot(a, w_ref[...], preferred_element_type=jnp.float32) + b_ref[...]
    z2_ref[...] = z2

    @pl.when(pl.program_id(0) == 0)
    def _():
        st2_ref[...] = jnp.zeros_like(st2_ref)

    st2_ref[0:1, :] += jnp.sum(z2, axis=0, keepdims=True)
    st2_ref[1:2, :] += jnp.sum(z2 * z2, axis=0, keepdims=True)


def _bnrelu_body(z_ref, st_ref, g_ref, be_ref, h_ref):
    mean = st_ref[0:1, :] * (1.0 / _N)
    var = st_ref[1:2, :] * (1.0 / _N) - mean * mean
    scale = g_ref[...] * lax.rsqrt(var + 1e-5)
    shift = be_ref[...] - mean * scale
    h_ref[...] = jnp.maximum(z_ref[...] * scale + shift, 0.0)


def _heads_body(x_ref, h1_ref, h2_ref, h3_ref, fw_ref, fb_ref, o_ref):
    logits = jnp.dot(x_ref[...], fw_ref[0], preferred_element_type=jnp.float32)
    logits += jnp.dot(h1_ref[...], fw_ref[1], preferred_element_type=jnp.float32)
    logits += jnp.dot(h2_ref[...], fw_ref[2], preferred_element_type=jnp.float32)
    logits += jnp.dot(h3_ref[...], fw_ref[3], preferred_element_type=jnp.float32)
    logits += jnp.sum(fb_ref[...], axis=0, keepdims=True)
    m = jnp.max(logits, axis=-1, keepdims=True)
    e = jnp.exp(logits - m)
    o_ref[...] = logits - m - jnp.log(jnp.sum(e, axis=-1, keepdims=True))


def _row_spec():
    return pl.BlockSpec((_BLK, _D), lambda i: (i, 0))


def _full_spec(shape):
    nd = len(shape)
    return pl.BlockSpec(shape, lambda i: (0,) * nd)


def _mm1(h, agg2, w, b):
    return pl.pallas_call(
        _mm1_body,
        grid=(_N // _BLK,),
        in_specs=[
            _row_spec(),
            pl.BlockSpec((_NC, _BLK, _D), lambda i: (0, i, 0)),
            _full_spec((_D, _D)),
            _full_spec((1, _D)),
        ],
        out_specs=[_row_spec(), _full_spec((8, _D))],
        out_shape=[
            jax.ShapeDtypeStruct((_N, _D), jnp.float32),
            jax.ShapeDtypeStruct((8, _D), jnp.float32),
        ],
    )(h, agg2, w, b)


def _mm2(z, st, g, be, w, b):
    return pl.pallas_call(
        _mm2_body,
        grid=(_N // _BLK,),
        in_specs=[
            _row_spec(),
            _full_spec((8, _D)),
            _full_spec((1, _D)),
            _full_spec((1, _D)),
            _full_spec((_D, _D)),
            _full_spec((1, _D)),
        ],
        out_specs=[_row_spec(), _full_spec((8, _D))],
        out_shape=[
            jax.ShapeDtypeStruct((_N, _D), jnp.float32),
            jax.ShapeDtypeStruct((8, _D), jnp.float32),
        ],
    )(z, st, g, be, w, b)


def _bnrelu(z, st, g, be):
    return pl.pallas_call(
        _bnrelu_body,
        grid=(_N // _BLK,),
        in_specs=[
            _row_spec(),
            _full_spec((8, _D)),
            _full_spec((1, _D)),
            _full_spec((1, _D)),
        ],
        out_specs=_row_spec(),
        out_shape=jax.ShapeDtypeStruct((_N, _D), jnp.float32),
    )(z, st, g, be)


def _heads(x, h1, h2, h3, fw, fb):
    return pl.pallas_call(
        _heads_body,
        grid=(_N // _BLK,),
        in_specs=[
            _row_spec(),
            _row_spec(),
            _row_spec(),
            _row_spec(),
            _full_spec((_L + 1, _D, _C)),
            _full_spec((_L + 1, _C)),
        ],
        out_specs=pl.BlockSpec((_BLK, _C), lambda i: (i, 0)),
        out_shape=jax.ShapeDtypeStruct((_N, _C), jnp.float32),
    )(x, h1, h2, h3, fw, fb)


def kernel(x, edge_index, cw1, cb1, cg1, cbe1, cw2, cb2, bng, bnb, fcw, fcb):
    npad = _EPAD - _E
    pad_i = jnp.arange(npad, dtype=jnp.int32)
    src = jnp.concatenate([edge_index[0], (pad_i * 37) % _N]).reshape(
        _NW, _NSLAB, _SLAB, _CH)
    dst = jnp.concatenate([edge_index[1], _N + pad_i % (_NPAD - _N)]).reshape(
        _NW, _NSLAB, _SLAB, _CH)
    zrows = jnp.zeros((640, _D), jnp.float32)

    h = x
    outs = [x]
    for l in range(_L):
        agg2 = _seg_sum_sc(h, src, dst, zrows)
        z1, st1 = _mm1(h, agg2, cw1[l], cb1[l].reshape(1, _D))
        z2, st2 = _mm2(z1, st1, cg1[l].reshape(1, _D), cbe1[l].reshape(1, _D),
                       cw2[l], cb2[l].reshape(1, _D))
        h = _bnrelu(z2, st2, bng[l].reshape(1, _D), bnb[l].reshape(1, _D))
        outs.append(h)

    out = _heads(outs[0], outs[1], outs[2], outs[3], fcw, fcb)
    return (out, 0)


# speedup vs baseline: 9.8900x; 1.0668x over previous
import functools

import jax
import jax.numpy as jnp
from jax import lax
from jax.experimental import pallas as pl
from jax.experimental.pallas import tpu as pltpu
from jax.experimental.pallas import tpu_sc as plsc

_N = 10000
_E = 320000
_D = 128
_C = 16
_L = 3

_NPAD = 10240
_CH = 128
_NC = 2
_NS = 16
_NW = _NC * _NS
_SLAB = 16
_NSLAB = 5
_CPW = _SLAB * _NSLAB
_EPAD = _NW * _CPW * _CH

_BLK = 1000


def _seg_sum_sc(h, src, dst, zrows):
    mesh = plsc.VectorSubcoreMesh(core_axis_name="c", subcore_axis_name="s")

    @functools.partial(
        pl.kernel,
        out_type=jax.ShapeDtypeStruct((_NC, _NPAD, _D), jnp.float32),
        mesh=mesh,
        scratch_types=[
            pltpu.VMEM((_SLAB, _CH), jnp.int32),
            pltpu.VMEM((_SLAB, _CH), jnp.int32),
            pltpu.VMEM((_SLAB, _CH), jnp.int32),
            pltpu.VMEM((_SLAB, _CH), jnp.int32),
            pltpu.VMEM((_CH, _D), jnp.float32),
            pltpu.VMEM((_CH, _D), jnp.float32),
            pltpu.VMEM_SHARED((_NPAD, _D), jnp.float32),
            pltpu.SemaphoreType.DMA,
            pltpu.SemaphoreType.DMA,
            pltpu.SemaphoreType.DMA,
        ],
    )
    def seg_kernel(h_hbm, src_hbm, dst_hbm, z_hbm, out_hbm,
                   src_s0, src_s1, dst_s0, dst_s1, rows0, rows1, acc,
                   semg0, semg1, semi):
        cid = lax.axis_index("c")
        sid = lax.axis_index("s")
        wid = sid * _NC + cid

        pltpu.async_copy(src_hbm.at[wid, 0], src_s0, semi)
        pltpu.async_copy(dst_hbm.at[wid, 0], dst_s0, semi)
        pltpu.sync_copy(z_hbm, acc.at[pl.ds(sid * 640, 640)])
        plsc.subcore_barrier()

        idx_bufs = [(src_s0, dst_s0), (src_s1, dst_s1)]
        for s in range(_NSLAB):
            ss, ds_ = idx_bufs[s % 2]
            ns, nd = idx_bufs[(s + 1) % 2]
            pltpu.make_async_copy(src_hbm.at[wid, s], ss, semi).wait()
            pltpu.make_async_copy(dst_hbm.at[wid, s], ds_, semi).wait()
            if s + 1 < _NSLAB:
                pltpu.async_copy(src_hbm.at[wid, s + 1], ns, semi)
                pltpu.async_copy(dst_hbm.at[wid, s + 1], nd, semi)

            pltpu.async_copy(h_hbm.at[ss.at[0]], rows0, semg0)

            def body(g, carry, ss=ss, ds_=ds_):
                j = 2 * g
                pltpu.async_copy(h_hbm.at[ss.at[j + 1]], rows1, semg1)
                pltpu.make_async_copy(h_hbm.at[ss.at[j]], rows0, semg0).wait()
                pltpu.sync_copy(rows0, acc.at[ds_.at[j]], add=True)

                @pl.when(g < _SLAB // 2 - 1)
                def _():
                    pltpu.async_copy(h_hbm.at[ss.at[j + 2]], rows0, semg0)

                pltpu.make_async_copy(h_hbm.at[ss.at[j + 1]], rows1,
                                      semg1).wait()
                pltpu.sync_copy(rows1, acc.at[ds_.at[j + 1]], add=True)
                return carry

            lax.fori_loop(0, _SLAB // 2, body, 0)

        plsc.subcore_barrier()

        pltpu.sync_copy(acc.at[pl.ds(sid * 640, 640)],
                        out_hbm.at[cid, pl.ds(sid * 640, 640)])

    return seg_kernel(h, src, dst, zrows)


def _bn_affine(st_ref, g_ref, be_ref):
    mean = st_ref[0:1, :] * (1.0 / _N)
    var = st_ref[1:2, :] * (1.0 / _N) - mean * mean
    scale = g_ref[...] * lax.rsqrt(var + 1e-5)
    shift = be_ref[...] - mean * scale
    return scale, shift


def _layer_body(h_ref, a_ref, w1_ref, b1_ref, g1_ref, be1_ref,
                w2_ref, b2_ref, g2_ref, be2_ref, ho_ref, z1_s, z2_s,
                st1, st2):
    p = pl.program_id(0)
    i = pl.program_id(1)

    @pl.when(jnp.logical_and(p == 0, i == 0))
    def _():
        st1[...] = jnp.zeros_like(st1)
        st2[...] = jnp.zeros_like(st2)

    @pl.when(p == 0)
    def _():
        u = h_ref[...] + a_ref[0] + a_ref[1]
        z = jnp.dot(u, w1_ref[...],
                    preferred_element_type=jnp.float32) + b1_ref[...]
        z1_s[pl.ds(i * _BLK, _BLK), :] = z
        st1[0:1, :] += jnp.sum(z, axis=0, keepdims=True)
        st1[1:2, :] += jnp.sum(z * z, axis=0, keepdims=True)

    @pl.when(p == 1)
    def _():
        scale, shift = _bn_affine(st1, g1_ref, be1_ref)
        a = jnp.maximum(z1_s[pl.ds(i * _BLK, _BLK), :] * scale + shift, 0.0)
        z2 = jnp.dot(a, w2_ref[...],
                     preferred_element_type=jnp.float32) + b2_ref[...]
        z2_s[pl.ds(i * _BLK, _BLK), :] = z2
        st2[0:1, :] += jnp.sum(z2, axis=0, keepdims=True)
        st2[1:2, :] += jnp.sum(z2 * z2, axis=0, keepdims=True)

    @pl.when(p == 2)
    def _():
        scale, shift = _bn_affine(st2, g2_ref, be2_ref)
        ho_ref[...] = jnp.maximum(
            z2_s[pl.ds(i * _BLK, _BLK), :] * scale + shift, 0.0)


def _heads_body(x_ref, h1_ref, h2_ref, h3_ref, fw_ref, fb_ref, o_ref):
    logits = jnp.dot(x_ref[...], fw_ref[0], preferred_element_type=jnp.float32)
    logits += jnp.dot(h1_ref[...], fw_ref[1], preferred_element_type=jnp.float32)
    logits += jnp.dot(h2_ref[...], fw_ref[2], preferred_element_type=jnp.float32)
    logits += jnp.dot(h3_ref[...], fw_ref[3], preferred_element_type=jnp.float32)
    logits += jnp.sum(fb_ref[...], axis=0, keepdims=True)
    m = jnp.max(logits, axis=-1, keepdims=True)
    e = jnp.exp(logits - m)
    o_ref[...] = logits - m - jnp.log(jnp.sum(e, axis=-1, keepdims=True))


def _row_spec():
    return pl.BlockSpec((_BLK, _D), lambda i: (i, 0))


def _full_spec(shape, ng=1):
    nd = len(shape)
    return pl.BlockSpec(shape, lambda *g: (0,) * nd)


def _gin_layer(h, agg2, w1, b1, g1, be1, w2, b2, g2, be2):
    p0row = pl.BlockSpec((_BLK, _D),
                         lambda p, i: (jnp.where(p == 0, i, 0), 0))
    return pl.pallas_call(
        _layer_body,
        grid=(3, _N // _BLK),
        in_specs=[
            p0row,
            pl.BlockSpec((_NC, _BLK, _D),
                         lambda p, i: (0, jnp.where(p == 0, i, 0), 0)),
            _full_spec((_D, _D)),
            _full_spec((1, _D)),
            _full_spec((1, _D)),
            _full_spec((1, _D)),
            _full_spec((_D, _D)),
            _full_spec((1, _D)),
            _full_spec((1, _D)),
            _full_spec((1, _D)),
        ],
        out_specs=pl.BlockSpec((_BLK, _D),
                               lambda p, i: (jnp.where(p == 2, i, 0), 0)),
        out_shape=jax.ShapeDtypeStruct((_N, _D), jnp.float32),
        scratch_shapes=[
            pltpu.VMEM((_N, _D), jnp.float32),
            pltpu.VMEM((_N, _D), jnp.float32),
            pltpu.VMEM((8, _D), jnp.float32),
            pltpu.VMEM((8, _D), jnp.float32),
        ],
    )(h, agg2, w1, b1, g1, be1, w2, b2, g2, be2)


def _heads(x, h1, h2, h3, fw, fb):
    return pl.pallas_call(
        _heads_body,
        grid=(_N // _BLK,),
        in_specs=[
            _row_spec(),
            _row_spec(),
            _row_spec(),
            _row_spec(),
            _full_spec((_L + 1, _D, _C)),
            _full_spec((_L + 1, _C)),
        ],
        out_specs=pl.BlockSpec((_BLK, _C), lambda i: (i, 0)),
        out_shape=jax.ShapeDtypeStruct((_N, _C), jnp.float32),
    )(x, h1, h2, h3, fw, fb)


def kernel(x, edge_index, cw1, cb1, cg1, cbe1, cw2, cb2, bng, bnb, fcw, fcb):
    npad = _EPAD - _E
    pad_i = jnp.arange(npad, dtype=jnp.int32)
    src = jnp.concatenate([edge_index[0], (pad_i * 37) % _N]).reshape(
        _NW, _NSLAB, _SLAB, _CH)
    dst = jnp.concatenate([edge_index[1], _N + pad_i % (_NPAD - _N)]).reshape(
        _NW, _NSLAB, _SLAB, _CH)
    zrows = jnp.zeros((640, _D), jnp.float32)

    h = x
    outs = [x]
    for l in range(_L):
        agg2 = _seg_sum_sc(h, src, dst, zrows)
        h = _gin_layer(h, agg2, cw1[l], cb1[l].reshape(1, _D),
                       cg1[l].reshape(1, _D), cbe1[l].reshape(1, _D),
                       cw2[l], cb2[l].reshape(1, _D),
                       bng[l].reshape(1, _D), bnb[l].reshape(1, _D))
        outs.append(h)

    out = _heads(outs[0], outs[1], outs[2], outs[3], fcw, fcb)
    return (out, 0)
